# Initial kernel scaffold; baseline (speedup 1.0000x reference)
import jax
import jax.numpy as jnp
from jax.experimental import pallas as pl


def kernel(x, edge_index, W_pre, b_pre, Wl1, Wr1, b1, Wl2, Wr2, b2):
    raise NotImplementedError("write your pallas kernel here")


# speedup vs baseline: 6.0757x; 6.0757x over previous
import functools

import jax
import jax.numpy as jnp
from jax import lax
from jax.experimental import pallas as pl
from jax.experimental.pallas import tpu as pltpu
from jax.experimental.pallas import tpu_sc as plsc

N = 10000
E = 320000
D = 128
NC, NS = 2, 16
NW = NC * NS
EPW = E // NW
CH = 40
NCH = EPW // CH
NKC = 10
KC = NCH // NKC
RPT = N // NS
DG = 16

_MESH = plsc.VectorSubcoreMesh(core_axis_name="c", subcore_axis_name="s")

def _sc_segment_sum():
    scratch = [
        pltpu.VMEM((KC, CH), jnp.int32),
        pltpu.VMEM((KC, CH), jnp.int32),
        pltpu.VMEM((CH, D), jnp.float32),
        pltpu.VMEM((CH, D), jnp.float32),
        pltpu.VMEM_SHARED((N, D), jnp.float32),
        pltpu.SemaphoreType.DMA,
        pltpu.SemaphoreType.DMA,
    ]

    def body(src_hbm, dst_hbm, g_hbm, acc_out,
             src_v, dst_v, buf0, buf1, acc_s, sem0, sem1):
        cid = lax.axis_index("c")
        sid = lax.axis_index("s")
        wid = sid * NC + cid
        base = sid * RPT
        zvec = jnp.zeros((16,), jnp.float32)

        def zrow(i, _):
            for jj in range(D // 16):
                buf0[i, pl.ds(jj * 16, 16)] = zvec
            return 0
        lax.fori_loop(0, CH, zrow, 0)
        nfull = RPT // CH
        rem = RPT - nfull * CH
        for k in range(nfull):
            pltpu.sync_copy(buf0, acc_s.at[pl.ds(base + k * CH, CH)])
        pltpu.sync_copy(buf0.at[pl.ds(0, rem)],
                        acc_s.at[pl.ds(base + nfull * CH, rem)])

        plsc.subcore_barrier()

        def fire(j, buf, sem):
            pltpu.async_copy(g_hbm.at[src_v.at[j]], buf, sem)

        def wait(j, buf, sem):
            pltpu.make_async_copy(g_hbm.at[src_v.at[j]], buf, sem).wait()

        def scatter(j, buf):
            pltpu.sync_copy(buf, acc_s.at[dst_v.at[j]], add=True)

        def chunk(k, _):
            pltpu.sync_copy(src_hbm.at[wid, k], src_v)
            pltpu.sync_copy(dst_hbm.at[wid, k], dst_v)
            fire(0, buf0, sem0)

            def step(i, _):
                j0 = 2 * i
                j1 = j0 + 1
                wait(j0, buf0, sem0)
                fire(j1, buf1, sem1)
                scatter(j0, buf0)
                wait(j1, buf1, sem1)
                fire(j1 + 1, buf0, sem0)
                scatter(j1, buf1)
                return 0
            lax.fori_loop(0, (KC - 1) // 2, step, 0)
            wait(KC - 1, buf0, sem0)
            scatter(KC - 1, buf0)
            return 0
        lax.fori_loop(0, NKC, chunk, 0)

        plsc.subcore_barrier()

        pltpu.sync_copy(acc_s.at[pl.ds(base, RPT)], acc_out.at[cid, sid])

    return pl.kernel(
        body, out_type=jax.ShapeDtypeStruct((NC, NS, RPT, D), jnp.float32),
        mesh=_MESH, scratch_types=tuple(scratch))


def _sc_degree():
    scratch = [
        pltpu.VMEM((KC, CH), jnp.int32),
        pltpu.VMEM((CH, DG), jnp.float32),
        pltpu.VMEM((CH, DG), jnp.float32),
        pltpu.VMEM_SHARED((N, DG), jnp.float32),
    ]

    def body(dst_hbm, deg_out, dst_v, ones_v, dzero, deg_s):
        cid = lax.axis_index("c")
        sid = lax.axis_index("s")
        wid = sid * NC + cid
        base = sid * RPT

        zvec = jnp.zeros((16,), jnp.float32)

        def zrow(i, _):
            dzero[i, :] = zvec
            ones_v[i, :] = zvec + 1.0
            return 0
        lax.fori_loop(0, CH, zrow, 0)
        nfull = RPT // CH
        rem = RPT - nfull * CH
        for k in range(nfull):
            pltpu.sync_copy(dzero, deg_s.at[pl.ds(base + k * CH, CH)])
        pltpu.sync_copy(dzero.at[pl.ds(0, rem)],
                        deg_s.at[pl.ds(base + nfull * CH, rem)])

        plsc.subcore_barrier()

        def chunk(k, _):
            pltpu.sync_copy(dst_hbm.at[wid, k], dst_v)

            def step(j, _):
                pltpu.sync_copy(ones_v, deg_s.at[dst_v.at[j]], add=True)
                return 0
            lax.fori_loop(0, KC, step, 0)
            return 0
        lax.fori_loop(0, NKC, chunk, 0)

        plsc.subcore_barrier()
        pltpu.sync_copy(deg_s.at[pl.ds(base, RPT)], deg_out.at[cid, sid])

    return pl.kernel(
        body, out_type=jax.ShapeDtypeStruct((NC, NS, RPT, DG), jnp.float32),
        mesh=_MESH, scratch_types=tuple(scratch))


_sc_pass = _sc_segment_sum()
_sc_deg = _sc_degree()

_R = 1000


def _row_spec():
    return pl.BlockSpec((_R, D), lambda i: (i, 0))


def _full_spec(shape):
    return pl.BlockSpec(shape, lambda i: tuple(0 for _ in shape))


def _stage_pre(x, W_pre, b_pre, Wl1, Wr1, b1):
    def body(x_r, wp_r, bp_r, wl_r, wr_r, b1_r, g_r, r_r):
        h = jnp.dot(x_r[...], wp_r[...],
                    preferred_element_type=jnp.float32) + bp_r[...]
        g_r[...] = jnp.dot(h, wl_r[...], preferred_element_type=jnp.float32)
        r_r[...] = jnp.dot(h, wr_r[...],
                           preferred_element_type=jnp.float32) + b1_r[...]

    return pl.pallas_call(
        body,
        grid=(N // _R,),
        in_specs=[_row_spec(), _full_spec((D, D)), _full_spec((1, D)),
                  _full_spec((D, D)), _full_spec((D, D)), _full_spec((1, D))],
        out_specs=[_row_spec(), _row_spec()],
        out_shape=[jax.ShapeDtypeStruct((N, D), jnp.float32),
                   jax.ShapeDtypeStruct((N, D), jnp.float32)],
    )(x, W_pre, b_pre.reshape(1, D), Wl1, Wr1, b1.reshape(1, D))


def _stage_mid(a0, a1, d0, d1, r0, Wl2, Wr2, b2):
    def body(a0_r, a1_r, d0_r, d1_r, r0_r, wl_r, wr_r, b2_r, g_r, r_r):
        deg = jnp.maximum(d0_r[:, 0:1] + d1_r[:, 0:1], 1.0)
        h = jnp.maximum((a0_r[...] + a1_r[...]) / deg + r0_r[...], 0.0)
        g_r[...] = jnp.dot(h, wl_r[...], preferred_element_type=jnp.float32)
        r_r[...] = jnp.dot(h, wr_r[...],
                           preferred_element_type=jnp.float32) + b2_r[...]

    dspec = pl.BlockSpec((_R, DG), lambda i: (i, 0))
    return pl.pallas_call(
        body,
        grid=(N // _R,),
        in_specs=[_row_spec(), _row_spec(), dspec, dspec, _row_spec(),
                  _full_spec((D, D)), _full_spec((D, D)), _full_spec((1, D))],
        out_specs=[_row_spec(), _row_spec()],
        out_shape=[jax.ShapeDtypeStruct((N, D), jnp.float32),
                   jax.ShapeDtypeStruct((N, D), jnp.float32)],
    )(a0, a1, d0, d1, r0, Wl2, Wr2, b2.reshape(1, D))


def _stage_out(a0, a1, d0, d1, r1):
    def body(a0_r, a1_r, d0_r, d1_r, r1_r, o_r):
        deg = jnp.maximum(d0_r[:, 0:1] + d1_r[:, 0:1], 1.0)
        h = (a0_r[...] + a1_r[...]) / deg + r1_r[...]
        nrm = jnp.sqrt(jnp.sum(h * h, axis=1, keepdims=True))
        o_r[...] = h / jnp.maximum(nrm, 1e-12)

    dspec = pl.BlockSpec((_R, DG), lambda i: (i, 0))
    return pl.pallas_call(
        body,
        grid=(N // _R,),
        in_specs=[_row_spec(), _row_spec(), dspec, dspec, _row_spec()],
        out_specs=_row_spec(),
        out_shape=jax.ShapeDtypeStruct((N, D), jnp.float32),
    )(a0, a1, d0, d1, r1)


def kernel(x, edge_index, W_pre, b_pre, Wl1, Wr1, b1, Wl2, Wr2, b2):
    ei = edge_index.astype(jnp.int32)
    src4d = ei[0].reshape(NW, NKC, KC, CH)
    dst4d = ei[1].reshape(NW, NKC, KC, CH)

    g0, r0 = _stage_pre(x, W_pre, b_pre, Wl1, Wr1, b1)
    deg = _sc_deg(dst4d)
    d0, d1 = deg[0].reshape(N, DG), deg[1].reshape(N, DG)
    a = _sc_pass(src4d, dst4d, g0)
    g1, r1 = _stage_mid(a[0].reshape(N, D), a[1].reshape(N, D),
                        d0, d1, r0, Wl2, Wr2, b2)
    b = _sc_pass(src4d, dst4d, g1)
    return _stage_out(b[0].reshape(N, D), b[1].reshape(N, D), d0, d1, r1)
